# baseline (speedup 1.0000x reference)
import functools

import jax
import jax.numpy as jnp
import numpy as np
from jax import lax
from jax.experimental import pallas as pl
from jax.experimental.pallas import tpu as pltpu
from jax.sharding import Mesh, PartitionSpec as P

try:
    from jax import shard_map as _shard_map
except ImportError:
    from jax.experimental.shard_map import shard_map as _shard_map

_VMEM_LIMIT = 56 * 1024 * 1024
_PAD = 64


def _interp_mat_t(out_size, in_size):
    if out_size == 1:
        src = jnp.zeros((out_size,), jnp.float32)
    else:
        src = jnp.arange(out_size, dtype=jnp.float32) * (
            (in_size - 1) / (out_size - 1))
    i0 = jnp.clip(jnp.floor(src), 0, in_size - 1).astype(jnp.int32)
    i1 = jnp.clip(i0 + 1, 0, in_size - 1)
    frac = src - i0.astype(jnp.float32)
    return (jax.nn.one_hot(i0, in_size, dtype=jnp.float32, axis=0)
            * (1.0 - frac)[None, :]
            + jax.nn.one_hot(i1, in_size, dtype=jnp.float32, axis=0)
            * frac[None, :])


def _refine_kernel(f_ref, pm_ref, wfs_ref, bfs_ref,
                   w11_ref, b11_ref, w12_ref, b12_ref,
                   w21_ref, b21_ref, w22_ref, b22_ref,
                   kup_ref, o_ref, xpad_f, xpad_c, *, H, W):
    HW = H * W
    col = lax.broadcasted_iota(jnp.int32, (1, HW), 1) % W
    mask_l = col != 0
    mask_r = col != (W - 1)

    def conv3x3(xpad, v_bf16, w_ref, b_ref):
        cin = v_bf16.shape[0]
        xpad[:, 0:_PAD] = jnp.zeros((cin, _PAD), jnp.bfloat16)
        xpad[:, _PAD + HW:] = jnp.zeros((cin, _PAD), jnp.bfloat16)
        xpad[:, _PAD:_PAD + HW] = v_bf16
        acc = None
        for t, (dy, dx) in enumerate((dy, dx) for dy in (-1, 0, 1)
                                     for dx in (-1, 0, 1)):
            s = dy * W + dx
            p = xpad[:, _PAD + s:_PAD + s + HW]
            if dx == -1:
                p = jnp.where(mask_l, p, jnp.bfloat16(0))
            elif dx == 1:
                p = jnp.where(mask_r, p, jnp.bfloat16(0))
            d = lax.dot_general(w_ref[t], p, (((0,), (0,)), ((), ())),
                                preferred_element_type=jnp.float32)
            acc = d if acc is None else acc + d
        return acc + b_ref[...]

    x = f_ref[0].astype(jnp.bfloat16)
    h1 = conv3x3(xpad_f, x, wfs_ref, bfs_ref)

    r = conv3x3(xpad_c, jnp.maximum(h1, 0.0).astype(jnp.bfloat16),
                w11_ref, b11_ref)
    r = conv3x3(xpad_c, jnp.maximum(r, 0.0).astype(jnp.bfloat16),
                w12_ref, b12_ref)
    s = h1 + r

    up = lax.dot_general(pm_ref[0], kup_ref[...], (((1,), (0,)), ((), ())),
                         preferred_element_type=jnp.float32)
    m = s + up

    r2 = conv3x3(xpad_c, jnp.maximum(m, 0.0).astype(jnp.bfloat16),
                 w21_ref, b21_ref)
    r2 = conv3x3(xpad_c, jnp.maximum(r2, 0.0).astype(jnp.bfloat16),
                 w22_ref, b22_ref)
    o_ref[0] = m + r2


def _run_fused(f_flat, pm_flat, wfs, bfs, w11, b11, w12, b12,
               w21, b21, w22, b22, kup, *, H, W):
    n, cin, HW = f_flat.shape
    _, c, hw = pm_flat.shape
    whole = lambda shp: pl.BlockSpec(shp, lambda i: (0,) * len(shp))
    return pl.pallas_call(
        functools.partial(_refine_kernel, H=H, W=W),
        out_shape=jax.ShapeDtypeStruct((n, c, HW), jnp.float32),
        grid=(n,),
        in_specs=[
            pl.BlockSpec((1, cin, HW), lambda i: (i, 0, 0)),
            pl.BlockSpec((1, c, hw), lambda i: (i, 0, 0)),
            whole((9, cin, c)), whole((c, 1)),
            whole((9, c, c)), whole((c, 1)),
            whole((9, c, c)), whole((c, 1)),
            whole((9, c, c)), whole((c, 1)),
            whole((9, c, c)), whole((c, 1)),
            whole((hw, HW)),
        ],
        out_specs=pl.BlockSpec((1, c, HW), lambda i: (i, 0, 0)),
        scratch_shapes=[
            pltpu.VMEM((cin, HW + 2 * _PAD), jnp.bfloat16),
            pltpu.VMEM((c, HW + 2 * _PAD), jnp.bfloat16),
        ],
        compiler_params=pltpu.CompilerParams(
            dimension_semantics=("arbitrary",),
            vmem_limit_bytes=_VMEM_LIMIT),
    )(f_flat, pm_flat, wfs, bfs, w11, b11, w12, b12, w21, b21, w22, b22, kup)


def kernel(f, pm, convFS_w, convFS_b,
           resFS_conv1_w, resFS_conv1_b, resFS_conv2_w, resFS_conv2_b,
           resMM_conv1_w, resMM_conv1_b, resMM_conv2_w, resMM_conv2_b):
    N, Cin, H, W = f.shape
    _, C, h, w = pm.shape
    HW, hw = H * W, h * w

    f_flat = f.reshape(N, Cin, HW)
    pm_flat = pm.reshape(N, C, hw)

    prep_w = lambda wc: wc.reshape(9, wc.shape[2], wc.shape[3]).astype(jnp.bfloat16)
    prep_b = lambda bc: bc.reshape(-1, 1)

    aht = _interp_mat_t(H, h)
    awt = _interp_mat_t(W, w)
    kup = (aht[:, None, :, None] * awt[None, :, None, :]).reshape(hw, HW)

    args = (f_flat, pm_flat,
            prep_w(convFS_w), prep_b(convFS_b),
            prep_w(resFS_conv1_w), prep_b(resFS_conv1_b),
            prep_w(resFS_conv2_w), prep_b(resFS_conv2_b),
            prep_w(resMM_conv1_w), prep_b(resMM_conv1_b),
            prep_w(resMM_conv2_w), prep_b(resMM_conv2_b),
            kup)

    run = functools.partial(_run_fused, H=H, W=W)
    devs = jax.devices()
    nd = len(devs)
    if nd > 1 and N % nd == 0:
        mesh = Mesh(np.array(devs), ("d",))
        out = _shard_map(run, mesh=mesh,
                         in_specs=(P("d"), P("d")) + (P(),) * 11,
                         out_specs=P("d"), check_vma=False)(*args)
    else:
        out = run(*args)
    return out.reshape(N, C, H, W)


# speedup vs baseline: 2.8935x; 2.8935x over previous
import functools

import jax
import jax.numpy as jnp
from jax import lax
from jax.experimental import pallas as pl
from jax.experimental.pallas import tpu as pltpu

_VMEM_LIMIT = 56 * 1024 * 1024
_PAD = 64


def _interp_mat_t(out_size, in_size):
    if out_size == 1:
        src = jnp.zeros((out_size,), jnp.float32)
    else:
        src = jnp.arange(out_size, dtype=jnp.float32) * (
            (in_size - 1) / (out_size - 1))
    i0 = jnp.clip(jnp.floor(src), 0, in_size - 1).astype(jnp.int32)
    i1 = jnp.clip(i0 + 1, 0, in_size - 1)
    frac = src - i0.astype(jnp.float32)
    return (jax.nn.one_hot(i0, in_size, dtype=jnp.float32, axis=0)
            * (1.0 - frac)[None, :]
            + jax.nn.one_hot(i1, in_size, dtype=jnp.float32, axis=0)
            * frac[None, :])


def _refine_kernel(f_ref, pm_ref, wfs_ref, bfs_ref,
                   w11_ref, b11_ref, w12_ref, b12_ref,
                   w21_ref, b21_ref, w22_ref, b22_ref,
                   kup_ref, o_ref, xpad_f, xpad_c, *, H, W):
    HW = H * W
    col = lax.broadcasted_iota(jnp.int32, (1, HW), 1) % W
    mask_l = col != 0
    mask_r = col != (W - 1)

    def conv3x3(xpad, v_bf16, w_ref, b_ref):
        cin = v_bf16.shape[0]
        xpad[:, 0:_PAD] = jnp.zeros((cin, _PAD), jnp.bfloat16)
        xpad[:, _PAD + HW:] = jnp.zeros((cin, _PAD), jnp.bfloat16)
        xpad[:, _PAD:_PAD + HW] = v_bf16
        acc = None
        for t, (dy, dx) in enumerate((dy, dx) for dy in (-1, 0, 1)
                                     for dx in (-1, 0, 1)):
            s = dy * W + dx
            p = xpad[:, _PAD + s:_PAD + s + HW]
            if dx == -1:
                p = jnp.where(mask_l, p, jnp.bfloat16(0))
            elif dx == 1:
                p = jnp.where(mask_r, p, jnp.bfloat16(0))
            d = lax.dot_general(w_ref[t], p, (((0,), (0,)), ((), ())),
                                preferred_element_type=jnp.float32)
            acc = d if acc is None else acc + d
        return acc + b_ref[...]

    x = f_ref[0].astype(jnp.bfloat16)
    h1 = conv3x3(xpad_f, x, wfs_ref, bfs_ref)

    r = conv3x3(xpad_c, jnp.maximum(h1, 0.0).astype(jnp.bfloat16),
                w11_ref, b11_ref)
    r = conv3x3(xpad_c, jnp.maximum(r, 0.0).astype(jnp.bfloat16),
                w12_ref, b12_ref)
    s = h1 + r

    up = lax.dot_general(pm_ref[0], kup_ref[...], (((1,), (0,)), ((), ())),
                         preferred_element_type=jnp.float32)
    m = s + up

    r2 = conv3x3(xpad_c, jnp.maximum(m, 0.0).astype(jnp.bfloat16),
                 w21_ref, b21_ref)
    r2 = conv3x3(xpad_c, jnp.maximum(r2, 0.0).astype(jnp.bfloat16),
                 w22_ref, b22_ref)
    o_ref[0] = m + r2


def _run_fused(f_flat, pm_flat, wfs, bfs, w11, b11, w12, b12,
               w21, b21, w22, b22, kup, *, H, W):
    n, cin, HW = f_flat.shape
    _, c, hw = pm_flat.shape
    whole = lambda shp: pl.BlockSpec(shp, lambda i: (0,) * len(shp))
    return pl.pallas_call(
        functools.partial(_refine_kernel, H=H, W=W),
        out_shape=jax.ShapeDtypeStruct((n, c, HW), jnp.float32),
        grid=(n,),
        in_specs=[
            pl.BlockSpec((1, cin, HW), lambda i: (i, 0, 0)),
            pl.BlockSpec((1, c, hw), lambda i: (i, 0, 0)),
            whole((9, cin, c)), whole((c, 1)),
            whole((9, c, c)), whole((c, 1)),
            whole((9, c, c)), whole((c, 1)),
            whole((9, c, c)), whole((c, 1)),
            whole((9, c, c)), whole((c, 1)),
            whole((hw, HW)),
        ],
        out_specs=pl.BlockSpec((1, c, HW), lambda i: (i, 0, 0)),
        scratch_shapes=[
            pltpu.VMEM((cin, HW + 2 * _PAD), jnp.bfloat16),
            pltpu.VMEM((c, HW + 2 * _PAD), jnp.bfloat16),
        ],
        compiler_params=pltpu.CompilerParams(
            dimension_semantics=("arbitrary",),
            vmem_limit_bytes=_VMEM_LIMIT),
    )(f_flat, pm_flat, wfs, bfs, w11, b11, w12, b12, w21, b21, w22, b22, kup)


def kernel(f, pm, convFS_w, convFS_b,
           resFS_conv1_w, resFS_conv1_b, resFS_conv2_w, resFS_conv2_b,
           resMM_conv1_w, resMM_conv1_b, resMM_conv2_w, resMM_conv2_b):
    N, Cin, H, W = f.shape
    _, C, h, w = pm.shape
    HW, hw = H * W, h * w

    f_flat = f.reshape(N, Cin, HW)
    pm_flat = pm.reshape(N, C, hw)

    prep_w = lambda wc: wc.reshape(9, wc.shape[2], wc.shape[3]).astype(jnp.bfloat16)
    prep_b = lambda bc: bc.reshape(-1, 1)

    aht = _interp_mat_t(H, h)
    awt = _interp_mat_t(W, w)
    kup = (aht[:, None, :, None] * awt[None, :, None, :]).reshape(hw, HW)

    args = (f_flat, pm_flat,
            prep_w(convFS_w), prep_b(convFS_b),
            prep_w(resFS_conv1_w), prep_b(resFS_conv1_b),
            prep_w(resFS_conv2_w), prep_b(resFS_conv2_b),
            prep_w(resMM_conv1_w), prep_b(resMM_conv1_b),
            prep_w(resMM_conv2_w), prep_b(resMM_conv2_b),
            kup)

    out = _run_fused(*args, H=H, W=W)
    return out.reshape(N, C, H, W)


# speedup vs baseline: 3.0100x; 1.0403x over previous
import functools

import jax
import jax.numpy as jnp
import numpy as np
from jax import lax
from jax.experimental import pallas as pl
from jax.experimental.pallas import tpu as pltpu

_VMEM_LIMIT = 56 * 1024 * 1024
_PAD = 64


def _interp_mat_np(out_size, in_size):
    if out_size == 1:
        src = np.zeros((out_size,), np.float64)
    else:
        src = np.arange(out_size, dtype=np.float64) * (
            (in_size - 1) / (out_size - 1))
    i0 = np.clip(np.floor(src).astype(np.int64), 0, in_size - 1)
    i1 = np.clip(i0 + 1, 0, in_size - 1)
    frac = (src - i0).astype(np.float32)
    m = np.zeros((out_size, in_size), np.float32)
    m[np.arange(out_size), i0] += 1.0 - frac
    m[np.arange(out_size), i1] += frac
    return m


def _refine_kernel(f_ref, pm_ref, wfs_ref, wrs_ref, b_ref, kup_ref,
                   o_ref, xpad_f, xpad_c, *, H, W):
    HW = H * W
    col = lax.broadcasted_iota(jnp.int32, (1, HW), 1) % W
    mask_l = col != 0
    mask_r = col != (W - 1)

    def conv3x3(xpad, v_bf16, w_ref, w_base, bias_col):
        cin = v_bf16.shape[0]
        xpad[:, 0:_PAD] = jnp.zeros((cin, _PAD), jnp.bfloat16)
        xpad[:, _PAD + HW:] = jnp.zeros((cin, _PAD), jnp.bfloat16)
        xpad[:, _PAD:_PAD + HW] = v_bf16
        acc = None
        for t, (dy, dx) in enumerate((dy, dx) for dy in (-1, 0, 1)
                                     for dx in (-1, 0, 1)):
            s = dy * W + dx
            p = xpad[:, _PAD + s:_PAD + s + HW]
            if dx == -1:
                p = jnp.where(mask_l, p, jnp.bfloat16(0))
            elif dx == 1:
                p = jnp.where(mask_r, p, jnp.bfloat16(0))
            d = lax.dot_general(w_ref[w_base + t], p, (((1,), (0,)), ((), ())),
                                preferred_element_type=jnp.float32)
            acc = d if acc is None else acc + d
        return acc + b_ref[:, bias_col:bias_col + 1]

    x = f_ref[0].astype(jnp.bfloat16)
    h1 = conv3x3(xpad_f, x, wfs_ref, 0, 0)

    r = conv3x3(xpad_c, jnp.maximum(h1, 0.0).astype(jnp.bfloat16), wrs_ref, 0, 1)
    r = conv3x3(xpad_c, jnp.maximum(r, 0.0).astype(jnp.bfloat16), wrs_ref, 9, 2)
    s = h1 + r

    up = lax.dot_general(pm_ref[0], kup_ref[...], (((1,), (0,)), ((), ())),
                         preferred_element_type=jnp.float32)
    m = s + up

    r2 = conv3x3(xpad_c, jnp.maximum(m, 0.0).astype(jnp.bfloat16), wrs_ref, 18, 3)
    r2 = conv3x3(xpad_c, jnp.maximum(r2, 0.0).astype(jnp.bfloat16), wrs_ref, 27, 4)
    o_ref[0] = (m + r2).reshape(o_ref.shape[1], H, W)


def kernel(f, pm, convFS_w, convFS_b,
           resFS_conv1_w, resFS_conv1_b, resFS_conv2_w, resFS_conv2_b,
           resMM_conv1_w, resMM_conv1_b, resMM_conv2_w, resMM_conv2_b):
    N, Cin, H, W = f.shape
    _, C, h, w = pm.shape
    HW, hw = H * W, h * w

    f_flat = f.reshape(N, Cin, HW)
    pm_flat = pm.reshape(N, C, hw)

    wfs = jnp.transpose(convFS_w.reshape(9, Cin, C), (0, 2, 1)).astype(jnp.bfloat16)
    wrs = jnp.transpose(
        jnp.stack([resFS_conv1_w, resFS_conv2_w, resMM_conv1_w, resMM_conv2_w])
        .reshape(4 * 9, C, C), (0, 2, 1)).astype(jnp.bfloat16)
    bcols = jnp.stack([convFS_b, resFS_conv1_b, resFS_conv2_b,
                       resMM_conv1_b, resMM_conv2_b], axis=1)

    aht = _interp_mat_np(H, h).T
    awt = _interp_mat_np(W, w).T
    kup = jnp.asarray(
        (aht[:, None, :, None] * awt[None, :, None, :]).reshape(hw, HW))

    whole = lambda shp: pl.BlockSpec(shp, lambda i: (0,) * len(shp))

    out = pl.pallas_call(
        functools.partial(_refine_kernel, H=H, W=W),
        out_shape=jax.ShapeDtypeStruct((N, C, H, W), jnp.float32),
        grid=(N,),
        in_specs=[
            pl.BlockSpec((1, Cin, HW), lambda i: (i, 0, 0)),
            pl.BlockSpec((1, C, hw), lambda i: (i, 0, 0)),
            whole((9, C, Cin)),
            whole((36, C, C)),
            whole((C, 5)),
            whole((hw, HW)),
        ],
        out_specs=pl.BlockSpec((1, C, H, W), lambda i: (i, 0, 0, 0)),
        scratch_shapes=[
            pltpu.VMEM((Cin, HW + 2 * _PAD), jnp.bfloat16),
            pltpu.VMEM((C, HW + 2 * _PAD), jnp.bfloat16),
        ],
        compiler_params=pltpu.CompilerParams(
            dimension_semantics=("arbitrary",),
            vmem_limit_bytes=_VMEM_LIMIT),
    )(f_flat, pm_flat, wfs, wrs, bcols, kup)
    return out


# speedup vs baseline: 4.2184x; 1.4015x over previous
import functools

import jax
import jax.numpy as jnp
import numpy as np
from jax import lax
from jax.experimental import pallas as pl
from jax.experimental.pallas import tpu as pltpu

_VMEM_LIMIT = 56 * 1024 * 1024
_PAD = 64


def _interp_mat_np(out_size, in_size):
    if out_size == 1:
        src = np.zeros((out_size,), np.float64)
    else:
        src = np.arange(out_size, dtype=np.float64) * (
            (in_size - 1) / (out_size - 1))
    i0 = np.clip(np.floor(src).astype(np.int64), 0, in_size - 1)
    i1 = np.clip(i0 + 1, 0, in_size - 1)
    frac = (src - i0).astype(np.float32)
    m = np.zeros((out_size, in_size), np.float32)
    m[np.arange(out_size), i0] += 1.0 - frac
    m[np.arange(out_size), i1] += frac
    return m


def _refine_kernel(f_ref, pm_ref, wfs_ref, wrs_ref, b_ref, kup_ref,
                   o_ref, *, H, W):
    HW = H * W
    col = lax.broadcasted_iota(jnp.int32, (1, HW), 1) % W
    mask_l = col != 0
    mask_r = col != (W - 1)

    def conv3x3(v_bf16, w_ref, w_base, bias_col):
        cin = v_bf16.shape[0]
        z1 = jnp.zeros((cin, 1), jnp.bfloat16)
        xm = jnp.where(mask_l, jnp.concatenate([z1, v_bf16[:, :HW - 1]], 1),
                       jnp.bfloat16(0))
        xp = jnp.where(mask_r, jnp.concatenate([v_bf16[:, 1:], z1], 1),
                       jnp.bfloat16(0))
        cols = (xm, v_bf16, xp)
        part = []
        for dy in range(3):
            acc = None
            for dx in range(3):
                d = lax.dot_general(w_ref[w_base + dy * 3 + dx], cols[dx],
                                    (((1,), (0,)), ((), ())),
                                    preferred_element_type=jnp.float32)
                acc = d if acc is None else acc + d
            part.append(acc)
        cout = part[0].shape[0]
        zw = jnp.zeros((cout, W), jnp.float32)
        out = part[1]
        out = out + jnp.concatenate([zw, part[0][:, :HW - W]], 1)
        out = out + jnp.concatenate([part[2][:, W:], zw], 1)
        return out + b_ref[:, bias_col:bias_col + 1]

    x = f_ref[0].astype(jnp.bfloat16)
    h1 = conv3x3(x, wfs_ref, 0, 0)

    r = conv3x3(jnp.maximum(h1, 0.0).astype(jnp.bfloat16), wrs_ref, 0, 1)
    r = conv3x3(jnp.maximum(r, 0.0).astype(jnp.bfloat16), wrs_ref, 9, 2)
    s = h1 + r

    up = lax.dot_general(pm_ref[0], kup_ref[...], (((1,), (0,)), ((), ())),
                         preferred_element_type=jnp.float32)
    m = s + up

    r2 = conv3x3(jnp.maximum(m, 0.0).astype(jnp.bfloat16), wrs_ref, 18, 3)
    r2 = conv3x3(jnp.maximum(r2, 0.0).astype(jnp.bfloat16), wrs_ref, 27, 4)
    o_ref[0] = m + r2


def kernel(f, pm, convFS_w, convFS_b,
           resFS_conv1_w, resFS_conv1_b, resFS_conv2_w, resFS_conv2_b,
           resMM_conv1_w, resMM_conv1_b, resMM_conv2_w, resMM_conv2_b):
    N, Cin, H, W = f.shape
    _, C, h, w = pm.shape
    HW, hw = H * W, h * w

    f_flat = f.reshape(N, Cin, HW)
    pm_flat = pm.reshape(N, C, hw)

    wfs = jnp.transpose(convFS_w.reshape(9, Cin, C), (0, 2, 1)).astype(jnp.bfloat16)
    wrs = jnp.transpose(
        jnp.stack([resFS_conv1_w, resFS_conv2_w, resMM_conv1_w, resMM_conv2_w])
        .reshape(4 * 9, C, C), (0, 2, 1)).astype(jnp.bfloat16)
    bcols = jnp.stack([convFS_b, resFS_conv1_b, resFS_conv2_b,
                       resMM_conv1_b, resMM_conv2_b], axis=1)

    aht = _interp_mat_np(H, h).T
    awt = _interp_mat_np(W, w).T
    kup = jnp.asarray(
        (aht[:, None, :, None] * awt[None, :, None, :]).reshape(hw, HW))

    whole = lambda shp: pl.BlockSpec(shp, lambda i: (0,) * len(shp))

    out = pl.pallas_call(
        functools.partial(_refine_kernel, H=H, W=W),
        out_shape=jax.ShapeDtypeStruct((N, C, HW), jnp.float32),
        grid=(N,),
        in_specs=[
            pl.BlockSpec((1, Cin, HW), lambda i: (i, 0, 0)),
            pl.BlockSpec((1, C, hw), lambda i: (i, 0, 0)),
            whole((9, C, Cin)),
            whole((36, C, C)),
            whole((C, 5)),
            whole((hw, HW)),
        ],
        out_specs=pl.BlockSpec((1, C, HW), lambda i: (i, 0, 0)),
        compiler_params=pltpu.CompilerParams(
            dimension_semantics=("arbitrary",),
            vmem_limit_bytes=_VMEM_LIMIT),
    )(f_flat, pm_flat, wfs, wrs, bcols, kup)
    return out.reshape(N, C, H, W)


# speedup vs baseline: 4.7997x; 1.1378x over previous
import functools

import jax
import jax.numpy as jnp
import numpy as np
from jax import lax
from jax.experimental import pallas as pl
from jax.experimental.pallas import tpu as pltpu

_VMEM_LIMIT = 56 * 1024 * 1024
_PAD = 64


def _interp_mat_np(out_size, in_size):
    if out_size == 1:
        src = np.zeros((out_size,), np.float64)
    else:
        src = np.arange(out_size, dtype=np.float64) * (
            (in_size - 1) / (out_size - 1))
    i0 = np.clip(np.floor(src).astype(np.int64), 0, in_size - 1)
    i1 = np.clip(i0 + 1, 0, in_size - 1)
    frac = (src - i0).astype(np.float32)
    m = np.zeros((out_size, in_size), np.float32)
    m[np.arange(out_size), i0] += 1.0 - frac
    m[np.arange(out_size), i1] += frac
    return m


def _refine_kernel(f_ref, pm_ref, wfs_ref, wrs_ref, b_ref, kup_ref,
                   o_ref, *, H, W):
    HW = H * W
    col = lax.broadcasted_iota(jnp.int32, (1, HW), 1) % W
    mask_l = col != 0
    mask_r = col != (W - 1)

    def conv3x3(v_bf16, w_ref, w_base, bias_col):
        cin = v_bf16.shape[0]
        z1 = jnp.zeros((cin, 1), jnp.bfloat16)
        xm = jnp.where(mask_l, jnp.concatenate([z1, v_bf16[:, :HW - 1]], 1),
                       jnp.bfloat16(0))
        xp = jnp.where(mask_r, jnp.concatenate([v_bf16[:, 1:], z1], 1),
                       jnp.bfloat16(0))
        cols = jnp.concatenate([xm, v_bf16, xp], axis=0)
        part = [lax.dot_general(w_ref[w_base + dy], cols,
                                (((1,), (0,)), ((), ())),
                                preferred_element_type=jnp.float32)
                for dy in range(3)]
        cout = part[0].shape[0]
        zw = jnp.zeros((cout, W), jnp.float32)
        out = part[1]
        out = out + jnp.concatenate([zw, part[0][:, :HW - W]], 1)
        out = out + jnp.concatenate([part[2][:, W:], zw], 1)
        return out + b_ref[:, bias_col:bias_col + 1]

    x = f_ref[0].astype(jnp.bfloat16)
    h1 = conv3x3(x, wfs_ref, 0, 0)

    r = conv3x3(jnp.maximum(h1, 0.0).astype(jnp.bfloat16), wrs_ref, 0, 1)
    r = conv3x3(jnp.maximum(r, 0.0).astype(jnp.bfloat16), wrs_ref, 3, 2)
    s = h1 + r

    up = lax.dot_general(pm_ref[0], kup_ref[...], (((1,), (0,)), ((), ())),
                         preferred_element_type=jnp.float32)
    m = s + up

    r2 = conv3x3(jnp.maximum(m, 0.0).astype(jnp.bfloat16), wrs_ref, 6, 3)
    r2 = conv3x3(jnp.maximum(r2, 0.0).astype(jnp.bfloat16), wrs_ref, 9, 4)
    o_ref[0] = m + r2


def kernel(f, pm, convFS_w, convFS_b,
           resFS_conv1_w, resFS_conv1_b, resFS_conv2_w, resFS_conv2_b,
           resMM_conv1_w, resMM_conv1_b, resMM_conv2_w, resMM_conv2_b):
    N, Cin, H, W = f.shape
    _, C, h, w = pm.shape
    HW, hw = H * W, h * w

    f_flat = f.reshape(N, Cin, HW)
    pm_flat = pm.reshape(N, C, hw)

    wfs = (jnp.transpose(convFS_w, (0, 3, 1, 2))
           .reshape(3, C, 3 * Cin).astype(jnp.bfloat16))
    wrs = (jnp.transpose(
        jnp.stack([resFS_conv1_w, resFS_conv2_w, resMM_conv1_w, resMM_conv2_w]),
        (0, 1, 4, 2, 3)).reshape(12, C, 3 * C).astype(jnp.bfloat16))
    bcols = jnp.stack([convFS_b, resFS_conv1_b, resFS_conv2_b,
                       resMM_conv1_b, resMM_conv2_b], axis=1)

    aht = _interp_mat_np(H, h).T
    awt = _interp_mat_np(W, w).T
    kup = jnp.asarray(
        (aht[:, None, :, None] * awt[None, :, None, :]).reshape(hw, HW))

    whole = lambda shp: pl.BlockSpec(shp, lambda i: (0,) * len(shp))

    out = pl.pallas_call(
        functools.partial(_refine_kernel, H=H, W=W),
        out_shape=jax.ShapeDtypeStruct((N, C, HW), jnp.float32),
        grid=(N,),
        in_specs=[
            pl.BlockSpec((1, Cin, HW), lambda i: (i, 0, 0)),
            pl.BlockSpec((1, C, hw), lambda i: (i, 0, 0)),
            whole((3, C, 3 * Cin)),
            whole((12, C, 3 * C)),
            whole((C, 5)),
            whole((hw, HW)),
        ],
        out_specs=pl.BlockSpec((1, C, HW), lambda i: (i, 0, 0)),
        compiler_params=pltpu.CompilerParams(
            dimension_semantics=("arbitrary",),
            vmem_limit_bytes=_VMEM_LIMIT),
    )(f_flat, pm_flat, wfs, wrs, bcols, kup)
    return out.reshape(N, C, H, W)


# speedup vs baseline: 4.9917x; 1.0400x over previous
import functools

import jax
import jax.numpy as jnp
import numpy as np
from jax import lax
from jax.experimental import pallas as pl
from jax.experimental.pallas import tpu as pltpu

_VMEM_LIMIT = 56 * 1024 * 1024
_PAD = 64


def _interp_mat_np(out_size, in_size):
    if out_size == 1:
        src = np.zeros((out_size,), np.float64)
    else:
        src = np.arange(out_size, dtype=np.float64) * (
            (in_size - 1) / (out_size - 1))
    i0 = np.clip(np.floor(src).astype(np.int64), 0, in_size - 1)
    i1 = np.clip(i0 + 1, 0, in_size - 1)
    frac = (src - i0).astype(np.float32)
    m = np.zeros((out_size, in_size), np.float32)
    m[np.arange(out_size), i0] += 1.0 - frac
    m[np.arange(out_size), i1] += frac
    return m


def _refine_kernel(f_ref, pm_ref, wfs_ref, wrs_ref, b_ref, kup_ref,
                   o_ref, *, H, W):
    HW = H * W
    col = lax.broadcasted_iota(jnp.int32, (1, HW), 1) % W
    mask_l = col != 0
    mask_r = col != (W - 1)

    def conv3x3(v_bf16, w2d, bias_col):
        cin = v_bf16.shape[0]
        z1 = jnp.zeros((cin, 1), jnp.bfloat16)
        xm = jnp.where(mask_l, jnp.concatenate([z1, v_bf16[:, :HW - 1]], 1),
                       jnp.bfloat16(0))
        xp = jnp.where(mask_r, jnp.concatenate([v_bf16[:, 1:], z1], 1),
                       jnp.bfloat16(0))
        cols = jnp.concatenate([xm, v_bf16, xp], axis=0)
        pall = lax.dot_general(w2d, cols, (((1,), (0,)), ((), ())),
                               preferred_element_type=jnp.float32)
        cout = pall.shape[0] // 3
        zw = jnp.zeros((cout, W), jnp.float32)
        out = pall[cout:2 * cout]
        out = out + jnp.concatenate([zw, pall[0:cout][:, :HW - W]], 1)
        out = out + jnp.concatenate([pall[2 * cout:][:, W:], zw], 1)
        return out + b_ref[:, bias_col:bias_col + 1]

    x = f_ref[0].astype(jnp.bfloat16)
    h1 = conv3x3(x, wfs_ref[...], 0)

    r = conv3x3(jnp.maximum(h1, 0.0).astype(jnp.bfloat16), wrs_ref[0], 1)
    r = conv3x3(jnp.maximum(r, 0.0).astype(jnp.bfloat16), wrs_ref[1], 2)
    s = h1 + r

    up = lax.dot_general(pm_ref[0], kup_ref[...], (((1,), (0,)), ((), ())),
                         preferred_element_type=jnp.float32)
    m = s + up

    r2 = conv3x3(jnp.maximum(m, 0.0).astype(jnp.bfloat16), wrs_ref[2], 3)
    r2 = conv3x3(jnp.maximum(r2, 0.0).astype(jnp.bfloat16), wrs_ref[3], 4)
    o_ref[0] = m + r2


def kernel(f, pm, convFS_w, convFS_b,
           resFS_conv1_w, resFS_conv1_b, resFS_conv2_w, resFS_conv2_b,
           resMM_conv1_w, resMM_conv1_b, resMM_conv2_w, resMM_conv2_b):
    N, Cin, H, W = f.shape
    _, C, h, w = pm.shape
    HW, hw = H * W, h * w

    f_flat = f.reshape(N, Cin, HW)
    pm_flat = pm.reshape(N, C, hw)

    wfs = (jnp.transpose(convFS_w, (0, 3, 1, 2))
           .reshape(3 * C, 3 * Cin).astype(jnp.bfloat16))
    wrs = (jnp.transpose(
        jnp.stack([resFS_conv1_w, resFS_conv2_w, resMM_conv1_w, resMM_conv2_w]),
        (0, 1, 4, 2, 3)).reshape(4, 3 * C, 3 * C).astype(jnp.bfloat16))
    bcols = jnp.stack([convFS_b, resFS_conv1_b, resFS_conv2_b,
                       resMM_conv1_b, resMM_conv2_b], axis=1)

    aht = _interp_mat_np(H, h).T
    awt = _interp_mat_np(W, w).T
    kup = jnp.asarray(
        (aht[:, None, :, None] * awt[None, :, None, :]).reshape(hw, HW))

    whole = lambda shp: pl.BlockSpec(shp, lambda i: (0,) * len(shp))

    out = pl.pallas_call(
        functools.partial(_refine_kernel, H=H, W=W),
        out_shape=jax.ShapeDtypeStruct((N, C, HW), jnp.float32),
        grid=(N,),
        in_specs=[
            pl.BlockSpec((1, Cin, HW), lambda i: (i, 0, 0)),
            pl.BlockSpec((1, C, hw), lambda i: (i, 0, 0)),
            whole((3 * C, 3 * Cin)),
            whole((4, 3 * C, 3 * C)),
            whole((C, 5)),
            whole((hw, HW)),
        ],
        out_specs=pl.BlockSpec((1, C, HW), lambda i: (i, 0, 0)),
        compiler_params=pltpu.CompilerParams(
            dimension_semantics=("arbitrary",),
            vmem_limit_bytes=_VMEM_LIMIT),
    )(f_flat, pm_flat, wfs, wrs, bcols, kup)
    return out.reshape(N, C, H, W)


# speedup vs baseline: 5.2039x; 1.0425x over previous
import functools

import jax
import jax.numpy as jnp
import numpy as np
from jax import lax
from jax.experimental import pallas as pl
from jax.experimental.pallas import tpu as pltpu

_VMEM_LIMIT = 100 * 1024 * 1024


def _interp_mat_np(out_size, in_size):
    if out_size == 1:
        src = np.zeros((out_size,), np.float64)
    else:
        src = np.arange(out_size, dtype=np.float64) * (
            (in_size - 1) / (out_size - 1))
    i0 = np.clip(np.floor(src).astype(np.int64), 0, in_size - 1)
    i1 = np.clip(i0 + 1, 0, in_size - 1)
    frac = (src - i0).astype(np.float32)
    m = np.zeros((out_size, in_size), np.float32)
    m[np.arange(out_size), i0] += 1.0 - frac
    m[np.arange(out_size), i1] += frac
    return m


def _refine_kernel(f_ref, pm_ref, wfs_ref, wrs_ref, b_ref, kup_ref,
                   o_ref, *, H, W, NI):
    HW = H * W
    L = NI * HW
    col = lax.broadcasted_iota(jnp.int32, (1, L), 1) % W
    mask_l = col != 0
    mask_r = col != (W - 1)

    def shift_down(p):
        cout = p.shape[0]
        zw = jnp.zeros((cout, W), jnp.float32)
        pieces = []
        for i in range(NI):
            pieces += [zw, p[:, i * HW:(i + 1) * HW - W]]
        return jnp.concatenate(pieces, 1)

    def shift_up(p):
        cout = p.shape[0]
        zw = jnp.zeros((cout, W), jnp.float32)
        pieces = []
        for i in range(NI):
            pieces += [p[:, i * HW + W:(i + 1) * HW], zw]
        return jnp.concatenate(pieces, 1)

    def conv3x3(v_bf16, w2d, bias_col):
        cin = v_bf16.shape[0]
        z1 = jnp.zeros((cin, 1), jnp.bfloat16)
        xm = jnp.where(mask_l, jnp.concatenate([z1, v_bf16[:, :L - 1]], 1),
                       jnp.bfloat16(0))
        xp = jnp.where(mask_r, jnp.concatenate([v_bf16[:, 1:], z1], 1),
                       jnp.bfloat16(0))
        cols = jnp.concatenate([xm, v_bf16, xp], axis=0)
        pall = lax.dot_general(w2d, cols, (((1,), (0,)), ((), ())),
                               preferred_element_type=jnp.float32)
        cout = pall.shape[0] // 3
        out = pall[cout:2 * cout]
        out = out + shift_down(pall[0:cout])
        out = out + shift_up(pall[2 * cout:])
        return out + b_ref[:, bias_col:bias_col + 1]

    x = jnp.concatenate([f_ref[i] for i in range(NI)], axis=1) \
        .astype(jnp.bfloat16) if NI > 1 else f_ref[0].astype(jnp.bfloat16)
    h1 = conv3x3(x, wfs_ref[...], 0)

    r = conv3x3(jnp.maximum(h1, 0.0).astype(jnp.bfloat16), wrs_ref[0], 1)
    r = conv3x3(jnp.maximum(r, 0.0).astype(jnp.bfloat16), wrs_ref[1], 2)
    s = h1 + r

    ups = [lax.dot_general(pm_ref[i], kup_ref[...], (((1,), (0,)), ((), ())),
                           preferred_element_type=jnp.float32)
           for i in range(NI)]
    m = s + (jnp.concatenate(ups, axis=1) if NI > 1 else ups[0])

    r2 = conv3x3(jnp.maximum(m, 0.0).astype(jnp.bfloat16), wrs_ref[2], 3)
    r2 = conv3x3(jnp.maximum(r2, 0.0).astype(jnp.bfloat16), wrs_ref[3], 4)
    out = m + r2
    for i in range(NI):
        o_ref[i] = out[:, i * HW:(i + 1) * HW]


def kernel(f, pm, convFS_w, convFS_b,
           resFS_conv1_w, resFS_conv1_b, resFS_conv2_w, resFS_conv2_b,
           resMM_conv1_w, resMM_conv1_b, resMM_conv2_w, resMM_conv2_b):
    N, Cin, H, W = f.shape
    _, C, h, w = pm.shape
    HW, hw = H * W, h * w
    NI = 2 if N % 2 == 0 else 1

    f_flat = f.reshape(N, Cin, HW)
    pm_flat = pm.reshape(N, C, hw)

    wfs = (jnp.transpose(convFS_w, (0, 3, 1, 2))
           .reshape(3 * C, 3 * Cin).astype(jnp.bfloat16))
    wrs = (jnp.transpose(
        jnp.stack([resFS_conv1_w, resFS_conv2_w, resMM_conv1_w, resMM_conv2_w]),
        (0, 1, 4, 2, 3)).reshape(4, 3 * C, 3 * C).astype(jnp.bfloat16))
    bcols = jnp.stack([convFS_b, resFS_conv1_b, resFS_conv2_b,
                       resMM_conv1_b, resMM_conv2_b], axis=1)

    aht = _interp_mat_np(H, h).T
    awt = _interp_mat_np(W, w).T
    kup = jnp.asarray(
        (aht[:, None, :, None] * awt[None, :, None, :]).reshape(hw, HW))

    whole = lambda shp: pl.BlockSpec(shp, lambda i: (0,) * len(shp))

    out = pl.pallas_call(
        functools.partial(_refine_kernel, H=H, W=W, NI=NI),
        out_shape=jax.ShapeDtypeStruct((N, C, HW), jnp.float32),
        grid=(N // NI,),
        in_specs=[
            pl.BlockSpec((NI, Cin, HW), lambda i: (i, 0, 0)),
            pl.BlockSpec((NI, C, hw), lambda i: (i, 0, 0)),
            whole((3 * C, 3 * Cin)),
            whole((4, 3 * C, 3 * C)),
            whole((C, 5)),
            whole((hw, HW)),
        ],
        out_specs=pl.BlockSpec((NI, C, HW), lambda i: (i, 0, 0)),
        compiler_params=pltpu.CompilerParams(
            dimension_semantics=("arbitrary",),
            vmem_limit_bytes=_VMEM_LIMIT),
    )(f_flat, pm_flat, wfs, wrs, bcols, kup)
    return out.reshape(N, C, H, W)
